# baseline (speedup 1.0000x reference)
import functools

import jax
import jax.numpy as jnp
from jax import lax
from jax.experimental import pallas as pl
from jax.experimental.pallas import tpu as pltpu
from jax.experimental.pallas import tpu_sc as plsc

_NC = 2
_NS = 16
_NW = _NC * _NS
_LANES = 16
_ROW = 128


def _sc_gather_combine(X, idx, indices, values_i):
    n, d = X.shape
    topk = indices.shape[1]
    bsz = idx.shape[0]
    spw = bsz // _NW
    ncol = d // _LANES
    gsz = _ROW // topk
    ngrp = spw // gsz
    nbuf = 3
    csz = 16
    nchk = spw // csz

    mesh = plsc.VectorSubcoreMesh(core_axis_name="c", subcore_axis_name="s")

    @functools.partial(
        pl.kernel,
        out_type=(
            jax.ShapeDtypeStruct((bsz, d), jnp.float32),
            jax.ShapeDtypeStruct((bsz, topk), jnp.int32),
        ),
        mesh=mesh,
        scratch_types=[
            pltpu.VMEM((spw,), jnp.int32),
            pltpu.VMEM((spw, topk), jnp.int32),
            pltpu.VMEM((spw, topk), jnp.int32),
            pltpu.VMEM((ngrp, _ROW), jnp.int32),
            pltpu.VMEM((nbuf, _ROW, d), jnp.float32),
            pltpu.VMEM((2 * _ROW // 32, d), jnp.float32),
            [pltpu.SemaphoreType.DMA] * nbuf,
            pltpu.SemaphoreType.DMA,
            pltpu.SemaphoreType.DMA,
        ],
        compiler_params=pltpu.CompilerParams(
            use_tc_tiling_on_sc=True, needs_layout_passes=False),
    )
    def sc_kernel(x_hbm, idx_hbm, ind_hbm, val_hbm, acc_hbm, nval_hbm,
                  idx_v, nidx_v, nval_v, nidx128_v, rows_v, bounce_v,
                  sems, gsem, asem):
        wid = lax.axis_index("s") * _NC + lax.axis_index("c")
        base = wid * spw

        pltpu.sync_copy(idx_hbm.at[pl.ds(base, spw)], idx_v)

        def fire_chunk(k):
            seed_ids = idx_v[pl.ds(k * csz, _LANES)]
            for lane in range(csz):
                r = seed_ids[lane]
                s = k * csz + lane
                pltpu.async_copy(ind_hbm.at[pl.ds(r, 1)],
                                 nidx_v.at[pl.ds(s, 1)], gsem)
                pltpu.async_copy(val_hbm.at[pl.ds(r, 1)],
                                 nval_v.at[pl.ds(s, 1)], gsem)

        def drain_chunk():
            for _ in range(2 * csz):
                pltpu.make_async_copy(ind_hbm.at[pl.ds(0, 1)],
                                      nidx_v.at[pl.ds(0, 1)], gsem).wait()

        fire_chunk(0)

        @pl.loop(1, nchk)
        def _chunks(k):
            drain_chunk()
            fire_chunk(k)

        drain_chunk()

        @pl.loop(0, ngrp)
        def _repack(q):
            for t in range(_ROW // _LANES):
                nidx128_v[q, pl.ds(t * _LANES, _LANES)] = (
                    nidx_v[q * gsz + t // (topk // _LANES),
                           pl.ds((t % (topk // _LANES)) * _LANES, _LANES)])

        def fire(g, buf):
            pltpu.async_copy(x_hbm.at[nidx128_v.at[g]], rows_v.at[buf],
                             sems[buf])

        for r in range(nbuf):
            fire(r, r)

        nit = -(-ngrp // nbuf) * nbuf

        @pl.loop(0, nit, step=nbuf)
        def _group_loop(g0):
            for r in range(nbuf):
                g = g0 + r

                @pl.when(g < ngrp)
                def _():
                    pltpu.make_async_copy(
                        x_hbm.at[nidx128_v.at[g]], rows_v.at[r],
                        sems[r]).wait()
                    vrow = rows_v.at[r]

                    @pl.when((g % 2 == 0) & (g > 0))
                    def _():
                        pltpu.make_async_copy(
                            bounce_v, acc_hbm.at[pl.ds(base, 2 * gsz)],
                            asem).wait()

                    @pl.loop(0, gsz)
                    def _seed(o):
                        s = g * gsz + o
                        accs = [jnp.zeros((_LANES,), jnp.float32)
                                for _ in range(ncol)]
                        for j in range(topk):
                            if j % _LANES == 0:
                                vals = plsc.bitcast(
                                    nval_v[s, pl.ds(j, _LANES)], jnp.float32)
                            w = jnp.full((_LANES,), vals[j % _LANES],
                                         dtype=jnp.float32)
                            for c in range(ncol):
                                accs[c] = accs[c] + w * vrow[
                                    o * topk + j, pl.ds(c * _LANES, _LANES)]
                        brow = (g % 2) * gsz + o
                        for c in range(ncol):
                            bounce_v[brow, pl.ds(c * _LANES, _LANES)] = accs[c]

                    @pl.when(g % 2 == 1)
                    def _():
                        off = pl.multiple_of(base + (g - 1) * gsz, 2 * gsz)
                        pltpu.async_copy(
                            bounce_v, acc_hbm.at[pl.ds(off, 2 * gsz)], asem)
                    nxt = g + nbuf

                    @pl.when(nxt < ngrp)
                    def _():
                        fire(nxt, r)

        pltpu.make_async_copy(
            bounce_v, acc_hbm.at[pl.ds(base, 2 * gsz)], asem).wait()
        pltpu.sync_copy(
            nval_v, nval_hbm.at[pl.ds(pl.multiple_of(base, 8), spw)])

    return sc_kernel(X, idx, indices, values_i)


def _tc_combine(acc, nval, W, b2):
    bsz, d = acc.shape
    topk = nval.shape[1]
    dout = W.shape[1]
    bm = 1024

    def body(a_ref, nv_ref, w_ref, b_ref, o_ref):
        s = jnp.sum(nv_ref[...], axis=1, keepdims=True)
        o_ref[...] = (
            jnp.dot(a_ref[...], w_ref[...], preferred_element_type=jnp.float32)
            + s * b_ref[...]
        )

    return pl.pallas_call(
        body,
        grid=(bsz // bm,),
        in_specs=[
            pl.BlockSpec((bm, d), lambda i: (i, 0)),
            pl.BlockSpec((bm, topk), lambda i: (i, 0)),
            pl.BlockSpec((d, dout), lambda i: (0, 0)),
            pl.BlockSpec((1, dout), lambda i: (0, 0)),
        ],
        out_specs=pl.BlockSpec((bm, dout), lambda i: (i, 0)),
        out_shape=jax.ShapeDtypeStruct((bsz, dout), jnp.float32),
    )(acc, nval, W, b2)


def kernel(X, idx, indices, values, W, b):
    values_i = lax.bitcast_convert_type(values, jnp.int32)
    acc, nval_i = _sc_gather_combine(X, idx, indices, values_i)
    nval = lax.bitcast_convert_type(nval_i, jnp.float32)
    return _tc_combine(acc, nval, W, b.reshape(1, -1))


# speedup vs baseline: 1.0905x; 1.0905x over previous
import functools

import jax
import jax.numpy as jnp
from jax import lax
from jax.experimental import pallas as pl
from jax.experimental.pallas import tpu as pltpu
from jax.experimental.pallas import tpu_sc as plsc

_NC = 2
_NS = 16
_NW = _NC * _NS
_LANES = 16
_ROW = 128


def _sc_gather_combine(X, idx, indices, values):
    n, d = X.shape
    topk = indices.shape[1]
    bsz = idx.shape[0]
    spw = bsz // _NW
    ncol = d // _LANES
    gsz = _ROW // topk
    ngrp = spw // gsz
    nbuf = 3
    csz = 16
    nchk = spw // csz

    mesh = plsc.VectorSubcoreMesh(core_axis_name="c", subcore_axis_name="s")

    @functools.partial(
        pl.kernel,
        out_type=(
            jax.ShapeDtypeStruct((bsz, d), jnp.float32),
            jax.ShapeDtypeStruct((bsz, topk), jnp.float32),
        ),
        mesh=mesh,
        scratch_types=[
            pltpu.VMEM((spw,), jnp.int32),
            pltpu.VMEM((spw, topk), jnp.int32),
            pltpu.VMEM((spw, topk), jnp.float32),
            pltpu.VMEM((ngrp, _ROW), jnp.int32),
            pltpu.VMEM((nbuf, _ROW, d), jnp.float32),
            pltpu.VMEM((2 * _ROW // 32, d), jnp.float32),
            [pltpu.SemaphoreType.DMA] * nbuf,
            pltpu.SemaphoreType.DMA,
            pltpu.SemaphoreType.DMA,
        ],
        compiler_params=pltpu.CompilerParams(
            use_tc_tiling_on_sc=True, needs_layout_passes=False),
    )
    def sc_kernel(x_hbm, idx_hbm, ind_hbm, val_hbm, acc_hbm, nval_hbm,
                  idx_v, nidx_v, nval_v, nidx128_v, rows_v, bounce_v,
                  sems, gsem, asem):
        wid = lax.axis_index("s") * _NC + lax.axis_index("c")
        base = wid * spw

        pltpu.sync_copy(idx_hbm.at[pl.ds(base, spw)], idx_v)

        def fire_chunk(k):
            seed_ids = idx_v[pl.ds(k * csz, _LANES)]
            for lane in range(csz):
                r = seed_ids[lane]
                s = k * csz + lane
                pltpu.async_copy(ind_hbm.at[pl.ds(r, 1)],
                                 nidx_v.at[pl.ds(s, 1)], gsem)
                pltpu.async_copy(val_hbm.at[pl.ds(r, 1)],
                                 nval_v.at[pl.ds(s, 1)], gsem)

        def drain_chunk():
            for _ in range(2 * csz):
                pltpu.make_async_copy(ind_hbm.at[pl.ds(0, 1)],
                                      nidx_v.at[pl.ds(0, 1)], gsem).wait()

        fire_chunk(0)

        @pl.loop(1, nchk)
        def _chunks(k):
            drain_chunk()
            fire_chunk(k)

        drain_chunk()

        @pl.loop(0, ngrp)
        def _repack(q):
            for t in range(_ROW // _LANES):
                nidx128_v[q, pl.ds(t * _LANES, _LANES)] = (
                    nidx_v[q * gsz + t // (topk // _LANES),
                           pl.ds((t % (topk // _LANES)) * _LANES, _LANES)])

        def fire(g, buf):
            pltpu.async_copy(x_hbm.at[nidx128_v.at[g]], rows_v.at[buf],
                             sems[buf])

        for r in range(nbuf):
            fire(r, r)

        nit = -(-ngrp // nbuf) * nbuf

        @pl.loop(0, nit, step=nbuf)
        def _group_loop(g0):
            for r in range(nbuf):
                g = g0 + r

                @pl.when(g < ngrp)
                def _():
                    pltpu.make_async_copy(
                        x_hbm.at[nidx128_v.at[g]], rows_v.at[r],
                        sems[r]).wait()
                    vrow = rows_v.at[r]

                    @pl.when((g % 2 == 0) & (g > 0))
                    def _():
                        pltpu.make_async_copy(
                            bounce_v, acc_hbm.at[pl.ds(base, 2 * gsz)],
                            asem).wait()

                    @pl.loop(0, gsz)
                    def _seed(o):
                        s = g * gsz + o
                        accs = [jnp.zeros((_LANES,), jnp.float32)
                                for _ in range(ncol)]
                        for j in range(topk):
                            if j % _LANES == 0:
                                vals = nval_v[s, pl.ds(j, _LANES)]
                            w = jnp.full((_LANES,), vals[j % _LANES],
                                         dtype=jnp.float32)
                            for c in range(ncol):
                                accs[c] = accs[c] + w * vrow[
                                    o * topk + j, pl.ds(c * _LANES, _LANES)]
                        brow = (g % 2) * gsz + o
                        for c in range(ncol):
                            bounce_v[brow, pl.ds(c * _LANES, _LANES)] = accs[c]

                    @pl.when(g % 2 == 1)
                    def _():
                        off = pl.multiple_of(base + (g - 1) * gsz, 2 * gsz)
                        pltpu.async_copy(
                            bounce_v, acc_hbm.at[pl.ds(off, 2 * gsz)], asem)
                    nxt = g + nbuf

                    @pl.when(nxt < ngrp)
                    def _():
                        fire(nxt, r)

        pltpu.make_async_copy(
            bounce_v, acc_hbm.at[pl.ds(base, 2 * gsz)], asem).wait()
        pltpu.sync_copy(
            nval_v, nval_hbm.at[pl.ds(pl.multiple_of(base, 8), spw)])

    return sc_kernel(X, idx, indices, values)


def _tc_combine(acc, nval, W, b2):
    bsz, d = acc.shape
    topk = nval.shape[1]
    dout = W.shape[1]
    bm = 1024

    def body(a_ref, nv_ref, w_ref, b_ref, o_ref):
        s = jnp.sum(nv_ref[...], axis=1, keepdims=True)
        o_ref[...] = (
            jnp.dot(a_ref[...], w_ref[...], preferred_element_type=jnp.float32)
            + s * b_ref[...]
        )

    return pl.pallas_call(
        body,
        grid=(bsz // bm,),
        in_specs=[
            pl.BlockSpec((bm, d), lambda i: (i, 0)),
            pl.BlockSpec((bm, topk), lambda i: (i, 0)),
            pl.BlockSpec((d, dout), lambda i: (0, 0)),
            pl.BlockSpec((1, dout), lambda i: (0, 0)),
        ],
        out_specs=pl.BlockSpec((bm, dout), lambda i: (i, 0)),
        out_shape=jax.ShapeDtypeStruct((bsz, dout), jnp.float32),
    )(acc, nval, W, b2)


def kernel(X, idx, indices, values, W, b):
    acc, nval = _sc_gather_combine(X, idx, indices, values)
    return _tc_combine(acc, nval, W, b.reshape(1, -1))


# speedup vs baseline: 1.1148x; 1.0223x over previous
import functools

import jax
import jax.numpy as jnp
from jax import lax
from jax.experimental import pallas as pl
from jax.experimental.pallas import tpu as pltpu
from jax.experimental.pallas import tpu_sc as plsc

_NC = 2
_NS = 16
_NW = _NC * _NS
_LANES = 16
_ROW = 128


def _sc_gather_combine(X, idx, indices, values):
    n, d = X.shape
    topk = indices.shape[1]
    bsz = idx.shape[0]
    spw = bsz // _NW
    ncol = d // _LANES
    gsz = _ROW // topk
    ngrp = spw // gsz
    nbuf = 3
    csz = 16
    nchk = spw // csz

    mesh = plsc.VectorSubcoreMesh(core_axis_name="c", subcore_axis_name="s")

    @functools.partial(
        pl.kernel,
        out_type=(
            jax.ShapeDtypeStruct((bsz, d), jnp.float32),
            jax.ShapeDtypeStruct((bsz, topk), jnp.float32),
        ),
        mesh=mesh,
        scratch_types=[
            pltpu.VMEM((spw,), jnp.int32),
            pltpu.VMEM((spw, topk), jnp.int32),
            pltpu.VMEM((spw, topk), jnp.float32),
            pltpu.VMEM((ngrp, _ROW), jnp.int32),
            pltpu.VMEM((nbuf, _ROW, d), jnp.float32),
            pltpu.VMEM((2 * _ROW // 32, d), jnp.float32),
            [pltpu.SemaphoreType.DMA] * nbuf,
            pltpu.SemaphoreType.DMA,
            pltpu.SemaphoreType.DMA,
        ],
        compiler_params=pltpu.CompilerParams(
            use_tc_tiling_on_sc=True, needs_layout_passes=False),
    )
    def sc_kernel(x_hbm, idx_hbm, ind_hbm, val_hbm, acc_hbm, nval_hbm,
                  idx_v, nidx_v, nval_v, nidx128_v, rows_v, bounce_v,
                  sems, gsem, asem):
        wid = lax.axis_index("s") * _NC + lax.axis_index("c")
        base = wid * spw

        pltpu.sync_copy(idx_hbm.at[pl.ds(base, spw)], idx_v)

        def fire_chunk(k):
            seed_ids = idx_v[pl.ds(k * csz, _LANES)]
            for lane in range(csz):
                r = seed_ids[lane]
                s = k * csz + lane
                pltpu.async_copy(ind_hbm.at[pl.ds(r, 1)],
                                 nidx_v.at[pl.ds(s, 1)], gsem)
                pltpu.async_copy(val_hbm.at[pl.ds(r, 1)],
                                 nval_v.at[pl.ds(s, 1)], gsem)

        def drain_chunk():
            for _ in range(2 * csz):
                pltpu.make_async_copy(ind_hbm.at[pl.ds(0, 1)],
                                      nidx_v.at[pl.ds(0, 1)], gsem).wait()

        gpc = csz // gsz

        def repack_chunk(k):
            @pl.loop(k * gpc, (k + 1) * gpc)
            def _repack(q):
                for t in range(_ROW // _LANES):
                    nidx128_v[q, pl.ds(t * _LANES, _LANES)] = (
                        nidx_v[q * gsz + t // (topk // _LANES),
                               pl.ds((t % (topk // _LANES)) * _LANES,
                                     _LANES)])

        def fire(g, buf):
            pltpu.async_copy(x_hbm.at[nidx128_v.at[g]], rows_v.at[buf],
                             sems[buf])

        fire_chunk(0)
        drain_chunk()
        repack_chunk(0)
        fire_chunk(1)
        for r in range(nbuf):
            fire(r, r)

        @pl.loop(2, nchk)
        def _chunks(k):
            drain_chunk()
            repack_chunk(k - 1)
            fire_chunk(k)

        drain_chunk()
        repack_chunk(nchk - 1)

        nit = -(-ngrp // nbuf) * nbuf

        @pl.loop(0, nit, step=nbuf)
        def _group_loop(g0):
            for r in range(nbuf):
                g = g0 + r

                @pl.when(g < ngrp)
                def _():
                    pltpu.make_async_copy(
                        x_hbm.at[nidx128_v.at[g]], rows_v.at[r],
                        sems[r]).wait()
                    vrow = rows_v.at[r]

                    @pl.when((g % 2 == 0) & (g > 0))
                    def _():
                        pltpu.make_async_copy(
                            bounce_v, acc_hbm.at[pl.ds(base, 2 * gsz)],
                            asem).wait()

                    @pl.loop(0, gsz)
                    def _seed(o):
                        s = g * gsz + o
                        accs = [jnp.zeros((_LANES,), jnp.float32)
                                for _ in range(ncol)]
                        for j in range(topk):
                            if j % _LANES == 0:
                                vals = nval_v[s, pl.ds(j, _LANES)]
                            w = jnp.full((_LANES,), vals[j % _LANES],
                                         dtype=jnp.float32)
                            for c in range(ncol):
                                accs[c] = accs[c] + w * vrow[
                                    o * topk + j, pl.ds(c * _LANES, _LANES)]
                        brow = (g % 2) * gsz + o
                        for c in range(ncol):
                            bounce_v[brow, pl.ds(c * _LANES, _LANES)] = accs[c]

                    @pl.when(g % 2 == 1)
                    def _():
                        off = pl.multiple_of(base + (g - 1) * gsz, 2 * gsz)
                        pltpu.async_copy(
                            bounce_v, acc_hbm.at[pl.ds(off, 2 * gsz)], asem)
                    nxt = g + nbuf

                    @pl.when(nxt < ngrp)
                    def _():
                        fire(nxt, r)

        pltpu.make_async_copy(
            bounce_v, acc_hbm.at[pl.ds(base, 2 * gsz)], asem).wait()
        pltpu.sync_copy(
            nval_v, nval_hbm.at[pl.ds(pl.multiple_of(base, 8), spw)])

    return sc_kernel(X, idx, indices, values)


def _tc_combine(acc, nval, W, b2):
    bsz, d = acc.shape
    topk = nval.shape[1]
    dout = W.shape[1]
    bm = 2048

    def body(a_ref, nv_ref, w_ref, b_ref, o_ref):
        s = jnp.sum(nv_ref[...], axis=1, keepdims=True)
        o_ref[...] = (
            jnp.dot(a_ref[...], w_ref[...], preferred_element_type=jnp.float32)
            + s * b_ref[...]
        )

    return pl.pallas_call(
        body,
        grid=(bsz // bm,),
        in_specs=[
            pl.BlockSpec((bm, d), lambda i: (i, 0)),
            pl.BlockSpec((bm, topk), lambda i: (i, 0)),
            pl.BlockSpec((d, dout), lambda i: (0, 0)),
            pl.BlockSpec((1, dout), lambda i: (0, 0)),
        ],
        out_specs=pl.BlockSpec((bm, dout), lambda i: (i, 0)),
        out_shape=jax.ShapeDtypeStruct((bsz, dout), jnp.float32),
    )(acc, nval, W, b2)


def kernel(X, idx, indices, values, W, b):
    acc, nval = _sc_gather_combine(X, idx, indices, values)
    return _tc_combine(acc, nval, W, b.reshape(1, -1))
